# baseline (speedup 1.0000x reference)
import functools

import jax
import jax.numpy as jnp
from jax import lax
from jax.experimental import pallas as pl
from jax.experimental.pallas import tpu as pltpu
from jax.experimental.pallas import tpu_sc as plsc

N_NODES = 10000
N_EDGES = 320000
D_FEAT = 128

NUM_WORKERS = 32
E_PER_W = N_EDGES // NUM_WORKERS
SUPER = 80
NSUP = E_PER_W // SUPER
STAGE_BLK = 80
NBLK = N_NODES // STAGE_BLK
KMAX = (NBLK + 15) // 16


def _gather_kernel(x_hbm, src_hbm, out_hbm,
                   x_sh, idx_all, rows0, rows1, rows2,
                   gsem0, gsem1, gsem2, ssem0, ssem1, ssem2, isem):
    cid = lax.axis_index("c")
    sid = lax.axis_index("s")
    wid = sid * 2 + cid
    base = wid * E_PER_W

    pltpu.async_copy(src_hbm.at[pl.ds(pl.multiple_of(base, 8), E_PER_W)],
                     idx_all, isem)

    bufs = (rows0, rows1)
    sems = (gsem0, gsem1)

    pltpu.async_copy(
        x_hbm.at[pl.ds(pl.multiple_of(sid * STAGE_BLK, 8), STAGE_BLK)],
        rows0, gsem0)

    for k in range(KMAX):
        b = k * 16 + sid
        buf_c, sem_c = bufs[k % 2], sems[k % 2]
        buf_n, sem_n = bufs[(k + 1) % 2], sems[(k + 1) % 2]

        @pl.when(b < NBLK)
        def _():
            pltpu.make_async_copy(
                x_hbm.at[pl.ds(0, STAGE_BLK)], buf_c, sem_c).wait()
            if k + 1 < KMAX:
                bn = (k + 1) * 16 + sid

                @pl.when(bn < NBLK)
                def _():
                    pltpu.async_copy(
                        x_hbm.at[pl.ds(pl.multiple_of(bn * STAGE_BLK, 8),
                                       STAGE_BLK)],
                        buf_n, sem_n)

            pltpu.sync_copy(
                buf_c, x_sh.at[pl.ds(pl.multiple_of(b * STAGE_BLK, 8),
                                     STAGE_BLK)])

    pltpu.make_async_copy(src_hbm.at[pl.ds(0, E_PER_W)], idx_all, isem).wait()
    plsc.subcore_barrier()

    rbufs = (rows0, rows1, rows2)
    gsems = (gsem0, gsem1, gsem2)
    ssems = (ssem0, ssem1, ssem2)

    def issue(i, rowsb, gsem):
        pltpu.async_copy(
            x_sh.at[idx_all.at[pl.ds(i * SUPER, SUPER)]], rowsb, gsem
        )

    def drain_gather(rowsb, gsem):
        pltpu.make_async_copy(x_hbm.at[pl.ds(0, SUPER)], rowsb, gsem).wait()

    def wait_store(rowsb, ssem):
        pltpu.make_async_copy(rowsb, out_hbm.at[pl.ds(0, SUPER)], ssem).wait()

    def step(i, r):
        c, n = r, (r + 1) % 3

        @pl.when(i + 1 < NSUP)
        def _():
            @pl.when(i >= 2)
            def _():
                wait_store(rbufs[n], ssems[n])
            issue(i + 1, rbufs[n], gsems[n])

        drain_gather(rbufs[c], gsems[c])
        off = pl.multiple_of(base + i * SUPER, 8)
        pltpu.async_copy(rbufs[c], out_hbm.at[pl.ds(off, SUPER)], ssems[c])

    issue(0, rows0, gsem0)

    def body(i, carry):
        for r in range(3):
            @pl.when(i % 3 == r)
            def _():
                step(i, r)

        return carry

    lax.fori_loop(0, NSUP, body, 0)

    for r in range(3):
        wait_store(rbufs[r], ssems[r])


def kernel(x, edge_index):
    src = edge_index.reshape(2 * N_EDGES)
    mesh = plsc.VectorSubcoreMesh(core_axis_name="c", subcore_axis_name="s")
    run = functools.partial(
        pl.kernel,
        out_type=jax.ShapeDtypeStruct((N_EDGES, D_FEAT), jnp.float32),
        mesh=mesh,
        scratch_types=[
            pltpu.VMEM_SHARED((N_NODES, D_FEAT), jnp.float32),
            pltpu.VMEM((E_PER_W,), jnp.int32),
            pltpu.VMEM((SUPER, D_FEAT), jnp.float32),
            pltpu.VMEM((SUPER, D_FEAT), jnp.float32),
            pltpu.VMEM((SUPER, D_FEAT), jnp.float32),
            pltpu.SemaphoreType.DMA,
            pltpu.SemaphoreType.DMA,
            pltpu.SemaphoreType.DMA,
            pltpu.SemaphoreType.DMA,
            pltpu.SemaphoreType.DMA,
            pltpu.SemaphoreType.DMA,
            pltpu.SemaphoreType.DMA,
        ],
    )(_gather_kernel)
    return run(x, src)


# speedup vs baseline: 1.0196x; 1.0196x over previous
import functools

import jax
import jax.numpy as jnp
from jax import lax
from jax.experimental import pallas as pl
from jax.experimental.pallas import tpu as pltpu
from jax.experimental.pallas import tpu_sc as plsc

N_NODES = 10000
N_EDGES = 320000
D_FEAT = 128

NUM_WORKERS = 32
E_PER_W = N_EDGES // NUM_WORKERS
SUPER = 160
NSUP = E_PER_W // SUPER
REM = E_PER_W - NSUP * SUPER
STAGE_BLK = 128
NBLK = N_NODES // STAGE_BLK
STAGE_REM = N_NODES - NBLK * STAGE_BLK


def _gather_kernel(x_hbm, src_hbm, out_hbm,
                   x_sh, idx_all, rows0, rows1,
                   gsem0, gsem1, ssem0, ssem1):
    cid = lax.axis_index("c")
    sid = lax.axis_index("s")
    wid = sid * 2 + cid
    base = wid * E_PER_W

    pltpu.sync_copy(src_hbm.at[pl.ds(pl.multiple_of(base, 8), E_PER_W)],
                    idx_all)

    KMAX = NBLK // 16 + 1
    bufs = (rows0, rows1)
    sems = (gsem0, gsem1)

    @pl.when(sid < NBLK)
    def _():
        pltpu.async_copy(
            x_hbm.at[pl.ds(pl.multiple_of(sid * STAGE_BLK, 8), STAGE_BLK)],
            rows0.at[pl.ds(0, STAGE_BLK)], gsem0)

    for k in range(KMAX):
        b = k * 16 + sid
        buf_c, sem_c = bufs[k % 2], sems[k % 2]
        buf_n, sem_n = bufs[(k + 1) % 2], sems[(k + 1) % 2]

        @pl.when(b < NBLK)
        def _():
            pltpu.make_async_copy(
                x_hbm.at[pl.ds(0, STAGE_BLK)],
                buf_c.at[pl.ds(0, STAGE_BLK)], sem_c).wait()
            if k + 1 < KMAX:
                bn = (k + 1) * 16 + sid

                @pl.when(bn < NBLK)
                def _():
                    pltpu.async_copy(
                        x_hbm.at[pl.ds(pl.multiple_of(bn * STAGE_BLK, 8),
                                       STAGE_BLK)],
                        buf_n.at[pl.ds(0, STAGE_BLK)], sem_n)

            pltpu.sync_copy(
                buf_c.at[pl.ds(0, STAGE_BLK)],
                x_sh.at[pl.ds(pl.multiple_of(b * STAGE_BLK, 8),
                              STAGE_BLK)])

    @pl.when(sid == 0)
    def _():
        r0 = pl.multiple_of(NBLK * STAGE_BLK, 8)
        pltpu.sync_copy(x_hbm.at[pl.ds(r0, STAGE_REM)],
                        rows0.at[pl.ds(0, STAGE_REM)])
        pltpu.sync_copy(rows0.at[pl.ds(0, STAGE_REM)],
                        x_sh.at[pl.ds(r0, STAGE_REM)])

    plsc.subcore_barrier()

    HG = SUPER // 2

    def issue(i, rowsb, gsem):
        for j in range(2):
            pltpu.async_copy(
                x_sh.at[idx_all.at[pl.ds(i * SUPER + j * HG, HG)]],
                rowsb.at[pl.ds(j * HG, HG)],
                gsem,
            )

    def drain_gather(rowsb, gsem):
        for j in range(2):
            pltpu.make_async_copy(
                x_hbm.at[pl.ds(0, HG)], rowsb.at[pl.ds(j * HG, HG)], gsem
            ).wait()

    def wait_store(rowsb, ssem):
        pltpu.make_async_copy(rowsb, out_hbm.at[pl.ds(0, SUPER)], ssem).wait()

    def step(i, rows_c, gsem_c, ssem_c, rows_n, gsem_n, ssem_n):
        @pl.when(i + 1 < NSUP)
        def _():
            @pl.when(i >= 1)
            def _():
                wait_store(rows_n, ssem_n)
            issue(i + 1, rows_n, gsem_n)

        drain_gather(rows_c, gsem_c)
        off = pl.multiple_of(base + i * SUPER, 8)
        pltpu.async_copy(rows_c, out_hbm.at[pl.ds(off, SUPER)], ssem_c)

    issue(0, rows0, gsem0)

    def body(i, carry):
        @pl.when(i % 2 == 0)
        def _():
            step(i, rows0, gsem0, ssem0, rows1, gsem1, ssem1)

        @pl.when(i % 2 == 1)
        def _():
            step(i, rows1, gsem1, ssem1, rows0, gsem0, ssem0)

        return carry

    lax.fori_loop(0, NSUP, body, 0)

    wait_store(rows0, ssem0)
    rem_off = pl.multiple_of(base + NSUP * SUPER, 8)
    pltpu.async_copy(
        x_sh.at[idx_all.at[pl.ds(NSUP * SUPER, REM)]],
        rows0.at[pl.ds(0, REM)],
        gsem0,
    )
    wait_store(rows1, ssem1)
    pltpu.make_async_copy(
        x_hbm.at[pl.ds(0, REM)], rows0.at[pl.ds(0, REM)], gsem0
    ).wait()
    pltpu.sync_copy(rows0.at[pl.ds(0, REM)], out_hbm.at[pl.ds(rem_off, REM)])


def kernel(x, edge_index):
    src = edge_index.reshape(2 * N_EDGES)
    mesh = plsc.VectorSubcoreMesh(core_axis_name="c", subcore_axis_name="s")
    run = functools.partial(
        pl.kernel,
        out_type=jax.ShapeDtypeStruct((N_EDGES, D_FEAT), jnp.float32),
        mesh=mesh,
        scratch_types=[
            pltpu.VMEM_SHARED((N_NODES, D_FEAT), jnp.float32),
            pltpu.VMEM((E_PER_W,), jnp.int32),
            pltpu.VMEM((SUPER, D_FEAT), jnp.float32),
            pltpu.VMEM((SUPER, D_FEAT), jnp.float32),
            pltpu.SemaphoreType.DMA,
            pltpu.SemaphoreType.DMA,
            pltpu.SemaphoreType.DMA,
            pltpu.SemaphoreType.DMA,
        ],
    )(_gather_kernel)
    return run(x, src)
